# baseline (speedup 1.0000x reference)
import jax
import jax.numpy as jnp
from jax.experimental import pallas as pl
from jax.experimental.pallas import tpu as pltpu

K_HOP = 2


def _dot(a, b):
    return jax.lax.dot_general(a, b, (((1,), (0,)), ((), ())),
                               preferred_element_type=jnp.float32)


def _dconv(z, Bn, N, A_q, A_h, W, bias):
    f = z.shape[1]
    fp = -(-f // 128) * 128
    if fp != f:
        z = jnp.concatenate(
            [z, jnp.zeros((z.shape[0], fp - f), z.dtype)], axis=1)
        f = fp
    blocks = [z[b * N:(b + 1) * N, :] for b in range(Bn)]
    zw = jnp.concatenate(blocks, axis=1)
    k1q = _dot(A_q, zw)
    k2q = _dot(A_q, k1q)
    k1h = _dot(A_h, zw)
    k2h = _dot(A_h, k1h)
    cat = jnp.concatenate([
        jnp.concatenate([blocks[b],
                         k1q[:, b * f:(b + 1) * f],
                         k2q[:, b * f:(b + 1) * f],
                         k1h[:, b * f:(b + 1) * f],
                         k2h[:, b * f:(b + 1) * f]], axis=1)
        for b in range(Bn)], axis=0)
    return _dot(cat, W) + bias


def _cell(x, h, Bn, N, A_q, A_h, Wg, bg, Wc, bc):
    H = h.shape[1]
    z = jnp.concatenate([x, h], axis=1)
    ru = jax.nn.sigmoid(_dconv(z, Bn, N, A_q, A_h, Wg, bg))
    r, u = ru[:, :H], ru[:, H:]
    zc = jnp.concatenate([x, r * h], axis=1)
    c = jnp.tanh(_dconv(zc, Bn, N, A_q, A_h, Wc, bc))
    return u * h + (1.0 - u) * c


def _body(adj_ref, adjT_ref, x_ref, gW1_ref, gb1_ref, gW2_ref, gb2_ref,
          eW0_ref, eb0_ref, eW1_ref, eb1_ref, eW2_ref, eb2_ref,
          eW3_ref, eb3_ref, eW4_ref, eb4_ref,
          eWg0_ref, ebg0_ref, eWc0_ref, ebc0_ref,
          eWg1_ref, ebg1_ref, eWc1_ref, ebc1_ref,
          dWg0_ref, dbg0_ref, dWc0_ref, dbc0_ref,
          dWg1_ref, dbg1_ref, dWc1_ref, dbc1_ref,
          fcW_ref, fcb_ref, out_ref, mT_scr, xw_scr):
    N = adj_ref.shape[0]
    BN, T = x_ref.shape
    Bn = BN // N
    H = ebc0_ref.shape[1]
    OUT = out_ref.shape[1]
    f32 = jnp.float32
    neg_inf = f32(-jnp.inf)
    pos_inf = f32(jnp.inf)

    adj = adj_ref[...]
    adjT = adjT_ref[...]

    dq = jnp.sum(adj, axis=1, keepdims=True)
    A_q = jnp.where(dq > 0, 1.0 / dq, 0.0) * adj
    dh = jnp.sum(adjT, axis=1, keepdims=True)
    A_h = jnp.where(dh > 0, 1.0 / dh, 0.0) * adjT

    ri = jax.lax.broadcasted_iota(jnp.int32, (N, N), 0)
    ci = jax.lax.broadcasted_iota(jnp.int32, (N, N), 1)
    diag = ri == ci
    adj_ns = jnp.where(diag, 0.0, adj)
    adj_nsT = jnp.where(diag, 0.0, adjT)
    maskf = (adj_ns > 0).astype(f32)
    maskTf = (adj_nsT > 0).astype(f32)
    degb = jnp.maximum(jnp.sum(maskf, axis=1, keepdims=True), 1.0)
    degw = jnp.maximum(jnp.sum(adj_ns, axis=1, keepdims=True), 1e-6)
    P = adj_ns / degw

    xr = x_ref[...]
    xwide = jnp.concatenate(
        [xr[b * N:(b + 1) * N, :] for b in range(Bn)], axis=1)
    mT_scr[...] = maskTf
    xw_scr[...] = xwide

    hg = jnp.maximum(_dot(xr, gW1_ref[...]) + gb1_ref[...], 0.0)
    logits = _dot(hg, gW2_ref[...]) + gb2_ref[...]
    E = logits.shape[1]
    io = jax.lax.broadcasted_iota(jnp.int32, (BN, E), 1)
    m1 = jnp.max(logits, axis=1, keepdims=True)
    i1 = jnp.min(jnp.where(logits == m1, io, E), axis=1, keepdims=True)
    ml = jnp.where(io == i1, neg_inf, logits)
    m2 = jnp.max(ml, axis=1, keepdims=True)
    i2 = jnp.min(jnp.where(ml == m2, io, E), axis=1, keepdims=True)
    t2 = jnp.exp(m2 - m1)
    g1 = 1.0 / (1.0 + t2)
    g2 = t2 * g1

    BT = Bn * T
    mean_w = _dot(maskf, xwide) / degb
    wmean_w = _dot(adj_ns, xwide) / degw
    diff_w = _dot(P, _dot(P, xwide))

    CH = 8

    def mm_body(jc, carry):
        mx, mn = carry
        mtb = mT_scr[pl.ds(jc * CH, CH), :]
        xc = xw_scr[pl.ds(jc * CH, CH), :]
        m3 = jax.lax.broadcast_in_dim(mtb, (CH, N, BT), (0, 1)) > 0
        x3 = jax.lax.broadcast_in_dim(xc, (CH, N, BT), (0, 2))
        mx = jnp.maximum(mx, jnp.max(jnp.where(m3, x3, neg_inf), axis=0))
        mn = jnp.minimum(mn, jnp.min(jnp.where(m3, x3, pos_inf), axis=0))
        return mx, mn

    mx0 = jnp.full((N, BT), neg_inf, f32)
    mn0 = jnp.full((N, BT), pos_inf, f32)
    mx, mn = jax.lax.fori_loop(0, N // CH, mm_body, (mx0, mn0))
    max_w = jnp.where(jnp.isfinite(mx), mx, 0.0)
    min_w = jnp.where(jnp.isfinite(mn), mn, 0.0)

    def _rows(w):
        return jnp.concatenate(
            [w[:, b * T:(b + 1) * T] for b in range(Bn)], axis=0)

    mean_agg = _rows(mean_w)
    wmean_agg = _rows(wmean_w)
    diff_agg = _rows(diff_w)
    max_agg = _rows(max_w)
    min_agg = _rows(min_w)

    aggs = (mean_agg, wmean_agg, max_agg, min_agg, diff_agg)
    eWs = (eW0_ref, eW1_ref, eW2_ref, eW3_ref, eW4_ref)
    ebs = (eb0_ref, eb1_ref, eb2_ref, eb3_ref, eb4_ref)
    combined = jnp.zeros((BN, T), f32)
    for e in range(E):
        oe = _dot(aggs[e], eWs[e][...]) + ebs[e][...]
        ge = (g1 * (i1 == e).astype(f32) + g2 * (i2 == e).astype(f32))
        combined = combined + ge * oe
    xf = jnp.where(xr == 0.0, combined, xr)

    enc0 = (eWg0_ref[...], ebg0_ref[...], eWc0_ref[...], ebc0_ref[...])
    enc1 = (eWg1_ref[...], ebg1_ref[...], eWc1_ref[...], ebc1_ref[...])
    dec0 = (dWg0_ref[...], dbg0_ref[...], dWc0_ref[...], dbc0_ref[...])
    dec1 = (dWg1_ref[...], dbg1_ref[...], dWc1_ref[...], dbc1_ref[...])
    fcb = fcb_ref[...]

    iotaT = jax.lax.broadcasted_iota(jnp.int32, (1, T), 1)
    iotaO = jax.lax.broadcasted_iota(jnp.int32, (1, OUT), 1)
    fcW_row = fcW_ref[...]

    def enc_body(t, hh):
        h0, h1 = hh
        xt = jnp.sum(jnp.where(iotaT == t, xf, 0.0),
                     axis=1, keepdims=True)
        h0n = _cell(xt, h0, Bn, N, A_q, A_h, *enc0)
        h1n = _cell(h0n, h1, Bn, N, A_q, A_h, *enc1)
        return (h0n, h1n)

    h0 = jnp.zeros((BN, H), f32)
    h1 = jnp.zeros((BN, H), f32)
    h0, h1 = jax.lax.fori_loop(0, T, enc_body, (h0, h1))

    inp0 = xf[:, T - 1:T]
    out_acc0 = jnp.zeros((BN, OUT), f32)

    def dec_body(t, carry):
        h0, h1, inp, acc = carry
        h0n = _cell(inp, h0, Bn, N, A_q, A_h, *dec0)
        h1n = _cell(h0n, h1, Bn, N, A_q, A_h, *dec1)
        out = jnp.sum(h1n * fcW_row, axis=1, keepdims=True) + fcb
        acc = jnp.where(iotaO == t, out, acc)
        return (h0n, h1n, out, acc)

    _, _, _, out_acc = jax.lax.fori_loop(
        0, OUT, dec_body, (h0, h1, inp0, out_acc0))
    out_ref[...] = out_acc


def kernel(adj, x_enc, x_t_mark, pos_mark, x_dec, gW1, gb1, gW2, gb2,
           expW0, expb0, expW1, expb1, expW2, expb2, expW3, expb3,
           expW4, expb4,
           enc_Wg0, enc_bg0, enc_Wc0, enc_bc0, enc_Wg1, enc_bg1,
           enc_Wc1, enc_bc1,
           dec_Wg0, dec_bg0, dec_Wc0, dec_bc0, dec_Wg1, dec_bg1,
           dec_Wc1, dec_bc1, fcW, fcb, epoch):
    B, T, N, _ = x_enc.shape
    OUT = x_dec.shape[1]
    xrows = jnp.transpose(x_enc[..., 0], (0, 2, 1)).reshape(B * N, T)
    adjT = jnp.transpose(adj)
    r2 = lambda v: v.reshape(1, -1)

    def padW(W):
        nmat = 2 * K_HOP + 1
        f = W.shape[0] // nmat
        fp = -(-f // 128) * 128
        if fp == f:
            return W
        W5 = W.reshape(nmat, f, W.shape[1])
        return jnp.pad(W5, ((0, 0), (0, fp - f), (0, 0))).reshape(
            nmat * fp, W.shape[1])
    args = (adj, adjT, xrows, gW1, r2(gb1), gW2, r2(gb2),
            expW0, r2(expb0), expW1, r2(expb1), expW2, r2(expb2),
            expW3, r2(expb3), expW4, r2(expb4),
            padW(enc_Wg0), r2(enc_bg0), padW(enc_Wc0), r2(enc_bc0),
            padW(enc_Wg1), r2(enc_bg1), padW(enc_Wc1), r2(enc_bc1),
            padW(dec_Wg0), r2(dec_bg0), padW(dec_Wc0), r2(dec_bc0),
            padW(dec_Wg1), r2(dec_bg1), padW(dec_Wc1), r2(dec_bc1),
            r2(fcW), r2(fcb))
    out = pl.pallas_call(
        _body,
        out_shape=jax.ShapeDtypeStruct((B * N, OUT), jnp.float32),
        scratch_shapes=[
            pltpu.VMEM((N, N), jnp.float32),
            pltpu.VMEM((N, B * T), jnp.float32),
        ],
        compiler_params=pltpu.CompilerParams(
            vmem_limit_bytes=100 * 1024 * 1024),
    )(*args)
    return jnp.transpose(out.reshape(B, N, OUT), (0, 2, 1))[..., None]


# speedup vs baseline: 1.1959x; 1.1959x over previous
import jax
import jax.numpy as jnp
from jax.experimental import pallas as pl
from jax.experimental.pallas import tpu as pltpu

K_HOP = 2


def _dot(a, b):
    return jax.lax.dot_general(a, b, (((1,), (0,)), ((), ())),
                               preferred_element_type=jnp.float32)


def _cell(x, h, Bn, N, A_q, A_h, Wg, bg, Wc, bc):
    H = h.shape[1]
    fin = x.shape[1]
    f = fin + H
    z = jnp.concatenate([x, h], axis=1)
    zb = [z[b * N:(b + 1) * N, :] for b in range(Bn)]
    zw = jnp.concatenate(zb, axis=1)
    gq1 = _dot(A_q, zw)
    gq2 = _dot(A_q, gq1)
    gh1 = _dot(A_h, zw)
    gh2 = _dot(A_h, gh1)
    cat_g = jnp.concatenate([
        jnp.concatenate([zb[b],
                         gq1[:, b * f:(b + 1) * f],
                         gq2[:, b * f:(b + 1) * f],
                         gh1[:, b * f:(b + 1) * f],
                         gh2[:, b * f:(b + 1) * f]], axis=1)
        for b in range(Bn)], axis=0)
    ru = jax.nn.sigmoid(_dot(cat_g, Wg) + bg)
    r, u = ru[:, :H], ru[:, H:]
    rh = r * h
    rhb = [rh[b * N:(b + 1) * N, :] for b in range(Bn)]
    rhw = jnp.concatenate(rhb, axis=1)
    cq1 = _dot(A_q, rhw)
    cq2 = _dot(A_q, cq1)
    ch1 = _dot(A_h, rhw)
    ch2 = _dot(A_h, ch1)
    cat_c = jnp.concatenate([
        jnp.concatenate([x[b * N:(b + 1) * N, :], rhb[b],
                         gq1[:, b * f:b * f + fin], cq1[:, b * H:(b + 1) * H],
                         gq2[:, b * f:b * f + fin], cq2[:, b * H:(b + 1) * H],
                         gh1[:, b * f:b * f + fin], ch1[:, b * H:(b + 1) * H],
                         gh2[:, b * f:b * f + fin], ch2[:, b * H:(b + 1) * H]],
                        axis=1)
        for b in range(Bn)], axis=0)
    c = jnp.tanh(_dot(cat_c, Wc) + bc)
    return u * h + (1.0 - u) * c


def _body(adj_ref, adjT_ref, x_ref, gW1_ref, gb1_ref, gW2_ref, gb2_ref,
          eW0_ref, eb0_ref, eW1_ref, eb1_ref, eW2_ref, eb2_ref,
          eW3_ref, eb3_ref, eW4_ref, eb4_ref,
          eWg0_ref, ebg0_ref, eWc0_ref, ebc0_ref,
          eWg1_ref, ebg1_ref, eWc1_ref, ebc1_ref,
          dWg0_ref, dbg0_ref, dWc0_ref, dbc0_ref,
          dWg1_ref, dbg1_ref, dWc1_ref, dbc1_ref,
          fcW_ref, fcb_ref, out_ref, mT_scr, xw_scr):
    N = adj_ref.shape[0]
    BN, T = x_ref.shape
    Bn = BN // N
    H = ebc0_ref.shape[1]
    OUT = out_ref.shape[1]
    f32 = jnp.float32
    neg_inf = f32(-jnp.inf)
    pos_inf = f32(jnp.inf)

    adj = adj_ref[...]
    adjT = adjT_ref[...]

    dq = jnp.sum(adj, axis=1, keepdims=True)
    A_q = jnp.where(dq > 0, 1.0 / dq, 0.0) * adj
    dh = jnp.sum(adjT, axis=1, keepdims=True)
    A_h = jnp.where(dh > 0, 1.0 / dh, 0.0) * adjT

    ri = jax.lax.broadcasted_iota(jnp.int32, (N, N), 0)
    ci = jax.lax.broadcasted_iota(jnp.int32, (N, N), 1)
    diag = ri == ci
    adj_ns = jnp.where(diag, 0.0, adj)
    adj_nsT = jnp.where(diag, 0.0, adjT)
    maskf = (adj_ns > 0).astype(f32)
    maskTf = (adj_nsT > 0).astype(f32)
    degb = jnp.maximum(jnp.sum(maskf, axis=1, keepdims=True), 1.0)
    degw = jnp.maximum(jnp.sum(adj_ns, axis=1, keepdims=True), 1e-6)
    P = adj_ns / degw

    xr = x_ref[...]
    xwide = jnp.concatenate(
        [xr[b * N:(b + 1) * N, :] for b in range(Bn)], axis=1)
    mT_scr[...] = maskTf
    xw_scr[...] = xwide

    hg = jnp.maximum(_dot(xr, gW1_ref[...]) + gb1_ref[...], 0.0)
    logits = _dot(hg, gW2_ref[...]) + gb2_ref[...]
    E = logits.shape[1]
    io = jax.lax.broadcasted_iota(jnp.int32, (BN, E), 1)
    m1 = jnp.max(logits, axis=1, keepdims=True)
    i1 = jnp.min(jnp.where(logits == m1, io, E), axis=1, keepdims=True)
    ml = jnp.where(io == i1, neg_inf, logits)
    m2 = jnp.max(ml, axis=1, keepdims=True)
    i2 = jnp.min(jnp.where(ml == m2, io, E), axis=1, keepdims=True)
    t2 = jnp.exp(m2 - m1)
    g1 = 1.0 / (1.0 + t2)
    g2 = t2 * g1

    BT = Bn * T
    mean_w = _dot(maskf, xwide) / degb
    wmean_w = _dot(adj_ns, xwide) / degw
    diff_w = _dot(P, _dot(P, xwide))

    CH = 8

    def mm_body(jc, carry):
        mx, mn = carry
        mtb = mT_scr[pl.ds(jc * CH, CH), :]
        xc = xw_scr[pl.ds(jc * CH, CH), :]
        m3 = jax.lax.broadcast_in_dim(mtb, (CH, N, BT), (0, 1)) > 0
        x3 = jax.lax.broadcast_in_dim(xc, (CH, N, BT), (0, 2))
        mx = jnp.maximum(mx, jnp.max(jnp.where(m3, x3, neg_inf), axis=0))
        mn = jnp.minimum(mn, jnp.min(jnp.where(m3, x3, pos_inf), axis=0))
        return mx, mn

    mx0 = jnp.full((N, BT), neg_inf, f32)
    mn0 = jnp.full((N, BT), pos_inf, f32)
    mx, mn = jax.lax.fori_loop(0, N // CH, mm_body, (mx0, mn0))
    max_w = jnp.where(jnp.isfinite(mx), mx, 0.0)
    min_w = jnp.where(jnp.isfinite(mn), mn, 0.0)

    def _rows(w):
        return jnp.concatenate(
            [w[:, b * T:(b + 1) * T] for b in range(Bn)], axis=0)

    mean_agg = _rows(mean_w)
    wmean_agg = _rows(wmean_w)
    diff_agg = _rows(diff_w)
    max_agg = _rows(max_w)
    min_agg = _rows(min_w)

    aggs = (mean_agg, wmean_agg, max_agg, min_agg, diff_agg)
    eWs = (eW0_ref, eW1_ref, eW2_ref, eW3_ref, eW4_ref)
    ebs = (eb0_ref, eb1_ref, eb2_ref, eb3_ref, eb4_ref)
    combined = jnp.zeros((BN, T), f32)
    for e in range(E):
        oe = _dot(aggs[e], eWs[e][...]) + ebs[e][...]
        ge = (g1 * (i1 == e).astype(f32) + g2 * (i2 == e).astype(f32))
        combined = combined + ge * oe
    xf = jnp.where(xr == 0.0, combined, xr)

    enc0 = (eWg0_ref[...], ebg0_ref[...], eWc0_ref[...], ebc0_ref[...])
    enc1 = (eWg1_ref[...], ebg1_ref[...], eWc1_ref[...], ebc1_ref[...])
    dec0 = (dWg0_ref[...], dbg0_ref[...], dWc0_ref[...], dbc0_ref[...])
    dec1 = (dWg1_ref[...], dbg1_ref[...], dWc1_ref[...], dbc1_ref[...])
    fcb = fcb_ref[...]

    iotaT = jax.lax.broadcasted_iota(jnp.int32, (1, T), 1)
    iotaO = jax.lax.broadcasted_iota(jnp.int32, (1, OUT), 1)
    fcW_row = fcW_ref[...]

    def enc_body(t, hh):
        h0, h1 = hh
        xt = jnp.sum(jnp.where(iotaT == t, xf, 0.0),
                     axis=1, keepdims=True)
        h0n = _cell(xt, h0, Bn, N, A_q, A_h, *enc0)
        h1n = _cell(h0n, h1, Bn, N, A_q, A_h, *enc1)
        return (h0n, h1n)

    h0 = jnp.zeros((BN, H), f32)
    h1 = jnp.zeros((BN, H), f32)
    h0, h1 = jax.lax.fori_loop(0, T, enc_body, (h0, h1))

    inp0 = xf[:, T - 1:T]
    out_acc0 = jnp.zeros((BN, OUT), f32)

    def dec_body(t, carry):
        h0, h1, inp, acc = carry
        h0n = _cell(inp, h0, Bn, N, A_q, A_h, *dec0)
        h1n = _cell(h0n, h1, Bn, N, A_q, A_h, *dec1)
        out = jnp.sum(h1n * fcW_row, axis=1, keepdims=True) + fcb
        acc = jnp.where(iotaO == t, out, acc)
        return (h0n, h1n, out, acc)

    _, _, _, out_acc = jax.lax.fori_loop(
        0, OUT, dec_body, (h0, h1, inp0, out_acc0))
    out_ref[...] = out_acc


def kernel(adj, x_enc, x_t_mark, pos_mark, x_dec, gW1, gb1, gW2, gb2,
           expW0, expb0, expW1, expb1, expW2, expb2, expW3, expb3,
           expW4, expb4,
           enc_Wg0, enc_bg0, enc_Wc0, enc_bc0, enc_Wg1, enc_bg1,
           enc_Wc1, enc_bc1,
           dec_Wg0, dec_bg0, dec_Wc0, dec_bc0, dec_Wg1, dec_bg1,
           dec_Wc1, dec_bc1, fcW, fcb, epoch):
    B, T, N, _ = x_enc.shape
    OUT = x_dec.shape[1]
    xrows = jnp.transpose(x_enc[..., 0], (0, 2, 1)).reshape(B * N, T)
    adjT = jnp.transpose(adj)
    r2 = lambda v: v.reshape(1, -1)
    args = (adj, adjT, xrows, gW1, r2(gb1), gW2, r2(gb2),
            expW0, r2(expb0), expW1, r2(expb1), expW2, r2(expb2),
            expW3, r2(expb3), expW4, r2(expb4),
            enc_Wg0, r2(enc_bg0), enc_Wc0, r2(enc_bc0),
            enc_Wg1, r2(enc_bg1), enc_Wc1, r2(enc_bc1),
            dec_Wg0, r2(dec_bg0), dec_Wc0, r2(dec_bc0),
            dec_Wg1, r2(dec_bg1), dec_Wc1, r2(dec_bc1),
            r2(fcW), r2(fcb))
    out = pl.pallas_call(
        _body,
        out_shape=jax.ShapeDtypeStruct((B * N, OUT), jnp.float32),
        scratch_shapes=[
            pltpu.VMEM((N, N), jnp.float32),
            pltpu.VMEM((N, B * T), jnp.float32),
        ],
        compiler_params=pltpu.CompilerParams(
            vmem_limit_bytes=100 * 1024 * 1024),
    )(*args)
    return jnp.transpose(out.reshape(B, N, OUT), (0, 2, 1))[..., None]
